# baseline (speedup 1.0000x reference)
import functools

import jax
import jax.numpy as jnp
from jax import lax
from jax.experimental import pallas as pl
from jax.experimental.pallas import tpu as pltpu
from jax.experimental.pallas import tpu_sc as plsc

NC = 2
NS = 16
L = 16
G = 128


@functools.lru_cache(maxsize=None)
def _build(F, V, D, B):
    TOT = B * F
    NW = NC * NS
    assert TOT % NW == 0
    PER_W = TOT // NW
    CH = 1664
    assert PER_W % CH == 0 and CH % F == 0 and CH % G == 0
    NCH = PER_W // CH
    NG = CH // G

    mesh = plsc.VectorSubcoreMesh(core_axis_name="c", subcore_axis_name="s")

    @functools.partial(
        pl.kernel,
        out_type=jax.ShapeDtypeStruct((TOT, D), jnp.float32),
        mesh=mesh,
        scratch_types=[
            pltpu.VMEM((CH,), jnp.int32),
            pltpu.VMEM((CH, D), jnp.float32),
            pltpu.SemaphoreType.DMA,
        ],
        compiler_params=pltpu.CompilerParams(use_tc_tiling_on_sc=False),
    )
    def gather_kernel(idx_hbm, tab_hbm, out_hbm, idx_v, rows_v, sem):
        wid = lax.axis_index("s") * NC + lax.axis_index("c")
        lanes = lax.iota(jnp.int32, L)

        def chunk_body(c, carry):
            base = wid * PER_W + c * CH
            pltpu.sync_copy(idx_hbm.at[pl.ds(base, CH)], idx_v)

            def fix(j, carry2):
                s = j * L
                f = (s + lanes) % F
                idx_v[pl.ds(s, L)] = idx_v[pl.ds(s, L)] + f * V
                return carry2

            lax.fori_loop(0, CH // L, fix, 0, unroll=4)

            copies = [
                pltpu.async_copy(
                    tab_hbm.at[idx_v.at[pl.ds(g * G, G)]],
                    rows_v.at[pl.ds(g * G, G)],
                    sem,
                )
                for g in range(NG)
            ]
            for cp in copies:
                cp.wait()
            pltpu.sync_copy(rows_v, out_hbm.at[pl.ds(base, CH)])
            return carry

        lax.fori_loop(0, NCH, chunk_body, 0)

    return gather_kernel


def kernel(indices, tables):
    B, F = indices.shape
    F2, V, D = tables.shape
    idx_flat = indices.reshape(-1)
    tab_flat = tables.reshape(F * V, D)
    out = _build(F, V, D, B)(idx_flat, tab_flat)
    return out.reshape(B, F * D)


# speedup vs baseline: 4.8089x; 4.8089x over previous
import functools

import jax
import jax.numpy as jnp
from jax import lax
from jax.experimental import pallas as pl
from jax.experimental.pallas import tpu as pltpu
from jax.experimental.pallas import tpu_sc as plsc

NC = 2
NS = 16
L = 16


@functools.lru_cache(maxsize=None)
def _build(F, V, D, B):
    ROWS = F * D
    NW = NC * NS
    assert ROWS % NW == 0
    PER_W = ROWS // NW
    HB = B // 2
    assert HB % L == 0

    mesh = plsc.VectorSubcoreMesh(core_axis_name="c", subcore_axis_name="s")

    @functools.partial(
        pl.kernel,
        out_type=jax.ShapeDtypeStruct((ROWS, B), jnp.float32),
        mesh=mesh,
        scratch_types=[
            pltpu.VMEM((V,), jnp.float32),
            pltpu.VMEM((HB,), jnp.int32),
            pltpu.VMEM((HB,), jnp.float32),
        ],
        compiler_params=pltpu.CompilerParams(needs_layout_passes=False),
    )
    def gather_kernel(idx_hbm, tab_hbm, out_hbm, row_v, idx_v, out_v):
        wid = lax.axis_index("s") * NC + lax.axis_index("c")

        def task(t, carry):
            c = wid * PER_W + t
            f = c // D
            d = c % D
            pltpu.sync_copy(tab_hbm.at[f, d, :], row_v)

            def half(h, carry2):
                pltpu.sync_copy(idx_hbm.at[f, pl.ds(h * HB, HB)], idx_v)

                def g(j, carry3):
                    s = j * L
                    ids = idx_v[pl.ds(s, L)]
                    out_v[pl.ds(s, L)] = plsc.load_gather(row_v, [ids])
                    return carry3

                lax.fori_loop(0, HB // L, g, 0, unroll=8)
                pltpu.sync_copy(out_v, out_hbm.at[c, pl.ds(h * HB, HB)])
                return carry2

            lax.fori_loop(0, 2, half, 0)
            return carry

        lax.fori_loop(0, PER_W, task, 0)

    return gather_kernel


def kernel(indices, tables):
    B, F = indices.shape
    F2, V, D = tables.shape
    idx_t = indices.T
    tab_t = jnp.transpose(tables, (0, 2, 1))
    out_t = _build(F, V, D, B)(idx_t, tab_t)
    return out_t.T.reshape(B, F * D)
